# Initial kernel scaffold; baseline (speedup 1.0000x reference)
import functools
import math

import jax
import jax.numpy as jnp
from jax import lax
from jax.experimental import pallas as pl
from jax.experimental.pallas import tpu as pltpu
from jax.experimental.pallas import tpu_sc as plsc

N = 10000
E = 160000
D = 512
C_CURV = 1.0
NCHUNK = 4
CHUNK = D // NCHUNK
BATCH = 128
NB = E // BATCH
NC = 2
NS = 16
ROWS_PER_TILE = N // NS

BN = 1000


def _linear_body(x_ref, w_ref, b_ref, ls_ref, hc_ref):
    h = lax.dot_general(x_ref[...], w_ref[...], (((1,), (1,)), ((), ())),
                        preferred_element_type=jnp.float32)
    h = h + b_ref[...]
    scale0 = jnp.exp(ls_ref[0, 0])
    time = jax.nn.sigmoid(h[:, 0:1]) * scale0 + (math.sqrt(C_CURV) + 0.5)
    x_narrow = h[:, 1:]
    sq = jnp.clip(jnp.sum(x_narrow * x_narrow, axis=-1, keepdims=True),
                  1e-8, None)
    sc = (time * time - C_CURV) / sq
    row = jnp.concatenate(
        [time, x_narrow * jnp.sqrt(jnp.clip(sc, 1e-8, None))], axis=-1)
    for cc in range(NCHUNK):
        hc_ref[cc, :, :] = row[:, cc * CHUNK:(cc + 1) * CHUNK]


def _lorentz_linear_chunked(x, W, b2, ls2):
    return pl.pallas_call(
        _linear_body,
        grid=(N // BN,),
        in_specs=[
            pl.BlockSpec((BN, D), lambda i: (i, 0)),
            pl.BlockSpec((D, D), lambda i: (0, 0)),
            pl.BlockSpec((1, D), lambda i: (0, 0)),
            pl.BlockSpec(memory_space=pltpu.SMEM),
        ],
        out_specs=pl.BlockSpec((NCHUNK, BN, CHUNK), lambda i: (0, i, 0)),
        out_shape=jax.ShapeDtypeStruct((NCHUNK, N, CHUNK), jnp.float32),
    )(x, W, b2, ls2)


def _norm_body(sc_ref, out_ref):
    s = sc_ref[...]
    total_sq = jnp.zeros((BN, 1), jnp.float32)
    for cc in range(NCHUNK):
        total_sq = total_sq + jnp.sum(s[cc] * s[cc], axis=-1, keepdims=True)
    s0 = s[0, :, 0:1]
    inner = total_sq - 2.0 * s0 * s0
    denom = jnp.sqrt(jnp.clip(jnp.abs(inner), 1e-8, None)) / math.sqrt(C_CURV)
    inv = 1.0 / denom
    for cc in range(NCHUNK):
        out_ref[:, cc * CHUNK:(cc + 1) * CHUNK] = s[cc] * inv


def _lorentz_normalize(sup_c):
    return pl.pallas_call(
        _norm_body,
        grid=(N // BN,),
        in_specs=[pl.BlockSpec((NCHUNK, BN, CHUNK), lambda i: (0, i, 0))],
        out_specs=pl.BlockSpec((BN, D), lambda i: (i, 0)),
        out_shape=jax.ShapeDtypeStruct((N, D), jnp.float32),
    )(sup_c)


def _aggregate_body(h2_hbm, src_hbm, dst_hbm, w_hbm, z_hbm, out_hbm,
                    acc, srcv, dstv, wv, idxv, rows, sem):
    core = lax.axis_index("c")
    sub = lax.axis_index("s")
    nb_tile = jnp.where(sub < NB - NS * (NB // NS), NB // NS + 1, NB // NS)

    for t in range(NCHUNK // NC):
        chunk = core * (NCHUNK // NC) + t
        row_off = chunk * N

        pltpu.sync_copy(z_hbm, acc.at[pl.ds(sub * ROWS_PER_TILE,
                                            ROWS_PER_TILE)])
        plsc.subcore_barrier()

        def batch_step(bi, _):
            b = sub + bi * NS
            pltpu.sync_copy(src_hbm.at[pl.ds(b, 1)], srcv)
            pltpu.sync_copy(dst_hbm.at[pl.ds(b, 1)], dstv)
            pltpu.sync_copy(w_hbm.at[pl.ds(b, 1)], wv)
            for i in range(BATCH // 16):
                idxv[0, pl.ds(i * 16, 16)] = (
                    srcv[0, pl.ds(i * 16, 16)] + row_off)
            pltpu.async_copy(h2_hbm.at[idxv.at[0]], rows, sem).wait()

            def mul_step(j, _):
                wj = wv[0, j]
                for i in range(CHUNK // 16):
                    sl = pl.ds(i * 16, 16)
                    rows[j, sl] = rows[j, sl] * wj
                return ()
            lax.fori_loop(0, BATCH, mul_step, (), unroll=2)
            pltpu.sync_copy(rows, acc.at[dstv.at[0]], add=True)
            return ()

        lax.fori_loop(0, nb_tile, batch_step, ())
        plsc.subcore_barrier()
        pltpu.sync_copy(
            acc.at[pl.ds(sub * ROWS_PER_TILE, ROWS_PER_TILE)],
            out_hbm.at[chunk, pl.ds(sub * ROWS_PER_TILE, ROWS_PER_TILE)])
        plsc.subcore_barrier()


def _aggregate(h2, src2, dst2, w2, z):
    mesh = plsc.VectorSubcoreMesh(core_axis_name="c", subcore_axis_name="s")
    kfn = pl.kernel(
        _aggregate_body,
        out_type=jax.ShapeDtypeStruct((NCHUNK, N, CHUNK), jnp.float32),
        mesh=mesh,
        scratch_types=[
            pltpu.VMEM_SHARED((N, CHUNK), jnp.float32),
            pltpu.VMEM((1, BATCH), jnp.int32),
            pltpu.VMEM((1, BATCH), jnp.int32),
            pltpu.VMEM((1, BATCH), jnp.float32),
            pltpu.VMEM((1, BATCH), jnp.int32),
            pltpu.VMEM((BATCH, CHUNK), jnp.float32),
            pltpu.SemaphoreType.DMA,
        ],
    )
    return kfn(h2, src2, dst2, w2, z)


@jax.jit
def kernel(x, edge_index, edge_weight, W, b, log_scale):
    b2 = b.reshape(1, D)
    ls2 = log_scale.reshape(1, 1)
    hc = _lorentz_linear_chunked(x, W, b2, ls2)
    h2 = hc.reshape(NCHUNK * N, CHUNK)
    src2 = edge_index[1].astype(jnp.int32).reshape(NB, BATCH)
    dst2 = edge_index[0].astype(jnp.int32).reshape(NB, BATCH)
    w2 = edge_weight.reshape(NB, BATCH)
    z = jnp.zeros((ROWS_PER_TILE, CHUNK), jnp.float32)
    sup_c = _aggregate(h2, src2, dst2, w2, z)
    return _lorentz_normalize(sup_c)


# speedup vs baseline: 2.8783x; 2.8783x over previous
import math

import jax
import jax.numpy as jnp
from jax import lax
from jax.experimental import pallas as pl
from jax.experimental.pallas import tpu as pltpu
from jax.experimental.pallas import tpu_sc as plsc

N = 10000
E = 160000
D = 512
C_CURV = 1.0
NCHUNK = 4
CHUNK = D // NCHUNK
BATCH = 128
NC = 2
NS = 16
NB_TILE = -(-E // (BATCH * NS))
E_PAD = NB_TILE * BATCH * NS
ROWS_TILE = 640
N_PAD = ROWS_TILE * NS

BN = 1000


def _linear_body(x_ref, w_ref, b_ref, ls_ref, *hc_refs):
    h = lax.dot_general(x_ref[...], w_ref[...], (((1,), (1,)), ((), ())),
                        preferred_element_type=jnp.float32)
    h = h + b_ref[...]
    scale0 = jnp.exp(ls_ref[0, 0])
    time = jax.nn.sigmoid(h[:, 0:1]) * scale0 + (math.sqrt(C_CURV) + 0.5)
    x_narrow = h[:, 1:]
    sq = jnp.clip(jnp.sum(x_narrow * x_narrow, axis=-1, keepdims=True),
                  1e-8, None)
    sc = (time * time - C_CURV) / sq
    row = jnp.concatenate(
        [time, x_narrow * jnp.sqrt(jnp.clip(sc, 1e-8, None))], axis=-1)
    for cc in range(NCHUNK):
        hc_refs[cc][...] = row[:, cc * CHUNK:(cc + 1) * CHUNK]


def _lorentz_linear_chunked(x, W, b2, ls2):
    return pl.pallas_call(
        _linear_body,
        grid=(N // BN,),
        in_specs=[
            pl.BlockSpec((BN, D), lambda i: (i, 0)),
            pl.BlockSpec((D, D), lambda i: (0, 0)),
            pl.BlockSpec((1, D), lambda i: (0, 0)),
            pl.BlockSpec(memory_space=pltpu.SMEM),
        ],
        out_specs=[pl.BlockSpec((BN, CHUNK), lambda i: (i, 0))
                   for _ in range(NCHUNK)],
        out_shape=[jax.ShapeDtypeStruct((N, CHUNK), jnp.float32)
                   for _ in range(NCHUNK)],
    )(x, W, b2, ls2)


def _norm_body(s0_ref, s1_ref, s2_ref, s3_ref, out_ref):
    refs = (s0_ref, s1_ref, s2_ref, s3_ref)
    total_sq = jnp.zeros((BN, 1), jnp.float32)
    for cc in range(NCHUNK):
        s = refs[cc][...]
        total_sq = total_sq + jnp.sum(s * s, axis=-1, keepdims=True)
    t0 = s0_ref[:, 0:1]
    inner = total_sq - 2.0 * t0 * t0
    denom = jnp.sqrt(jnp.clip(jnp.abs(inner), 1e-8, None)) / math.sqrt(C_CURV)
    inv = 1.0 / denom
    for cc in range(NCHUNK):
        out_ref[:, cc * CHUNK:(cc + 1) * CHUNK] = refs[cc][...] * inv


def _lorentz_normalize(sup):
    return pl.pallas_call(
        _norm_body,
        grid=(N // BN,),
        in_specs=[pl.BlockSpec((BN, CHUNK), lambda i: (i, 0))
                  for _ in range(NCHUNK)],
        out_specs=pl.BlockSpec((BN, D), lambda i: (i, 0)),
        out_shape=jax.ShapeDtypeStruct((N, D), jnp.float32),
    )(*sup)


def _aggregate_body(h0, h1, h2, h3, src_hbm, dst_hbm, w_hbm, z_hbm,
                    o0, o1, o2, o3,
                    acc, idx_l, dst_l, w_l, rows, sem):
    h_refs = (h0, h1, h2, h3)
    out_refs = (o0, o1, o2, o3)
    core = lax.axis_index("c")
    sub = lax.axis_index("s")

    pltpu.sync_copy(src_hbm.at[sub], idx_l)
    pltpu.sync_copy(dst_hbm.at[sub], dst_l)
    pltpu.sync_copy(w_hbm.at[sub], w_l)

    for chunk in range(NCHUNK):
        @pl.when(core == chunk // NC)
        def _pass():
            hk = h_refs[chunk]
            ok = out_refs[chunk]
            pltpu.sync_copy(z_hbm, acc.at[pl.ds(sub * ROWS_TILE, ROWS_TILE)])
            plsc.subcore_barrier()

            def batch_step(bi, _):
                pltpu.async_copy(hk.at[idx_l.at[bi]], rows, sem).wait()

                def mul_step(g, _):
                    wvec = w_l[bi, pl.ds(g * 16, 16)]
                    for l in range(16):
                        wj = wvec[l]
                        j = g * 16 + l
                        for i in range(CHUNK // 16):
                            sl = pl.ds(i * 16, 16)
                            rows[j, sl] = rows[j, sl] * wj
                    return ()
                lax.fori_loop(0, BATCH // 16, mul_step, ())
                pltpu.sync_copy(rows, acc.at[dst_l.at[bi]], add=True)
                return ()

            lax.fori_loop(0, NB_TILE, batch_step, ())
            plsc.subcore_barrier()
            pltpu.sync_copy(acc.at[pl.ds(sub * ROWS_TILE, ROWS_TILE)],
                            ok.at[pl.ds(sub * ROWS_TILE, ROWS_TILE)])
            plsc.subcore_barrier()


def _aggregate(hs, src3, dst3, w3, z):
    mesh = plsc.VectorSubcoreMesh(core_axis_name="c", subcore_axis_name="s")
    kfn = pl.kernel(
        _aggregate_body,
        out_type=[jax.ShapeDtypeStruct((N_PAD, CHUNK), jnp.float32)
                  for _ in range(NCHUNK)],
        mesh=mesh,
        scratch_types=[
            pltpu.VMEM_SHARED((N_PAD, CHUNK), jnp.float32),
            pltpu.VMEM((NB_TILE, BATCH), jnp.int32),
            pltpu.VMEM((NB_TILE, BATCH), jnp.int32),
            pltpu.VMEM((NB_TILE, BATCH), jnp.float32),
            pltpu.VMEM((BATCH, CHUNK), jnp.float32),
            pltpu.SemaphoreType.DMA,
        ],
    )
    return kfn(*hs, src3, dst3, w3, z)


@jax.jit
def kernel(x, edge_index, edge_weight, W, b, log_scale):
    b2 = b.reshape(1, D)
    ls2 = log_scale.reshape(1, 1)
    hs = _lorentz_linear_chunked(x, W, b2, ls2)

    pad = E_PAD - E
    src = jnp.concatenate(
        [edge_index[1].astype(jnp.int32), jnp.zeros((pad,), jnp.int32)])
    dst = jnp.concatenate(
        [edge_index[0].astype(jnp.int32), jnp.zeros((pad,), jnp.int32)])
    w = jnp.concatenate([edge_weight, jnp.zeros((pad,), jnp.float32)])
    src3 = src.reshape(NS, NB_TILE, BATCH)
    dst3 = dst.reshape(NS, NB_TILE, BATCH)
    w3 = w.reshape(NS, NB_TILE, BATCH)
    z = jnp.zeros((ROWS_TILE, CHUNK), jnp.float32)

    sup = _aggregate(hs, src3, dst3, w3, z)
    sup = [s[:N] for s in sup]
    return _lorentz_normalize(sup)
